# baseline (speedup 1.0000x reference)
import functools

import jax
import jax.numpy as jnp
from jax import lax
from jax.experimental import pallas as pl
from jax.experimental.pallas import tpu as pltpu
from jax.experimental.pallas import tpu_sc as plsc

N = 10000
K = 32
C = 128
E = N * K

_P = 5
_EC = E // _P
_NP = N // _P


_TN1 = 1000


def _k1_body(f_ref, c1w_ref, d0w_ref, d1w_ref, d0b_ref, d1b_ref,
             g_ref, wf_ref, bf_ref):
    f_blk = f_ref[...]
    g_ref[...] = lax.dot_general(
        f_blk, c1w_ref[...], (((1,), (1,)), ((), ())),
        preferred_element_type=jnp.float32)
    d0w = d0w_ref[...]
    d1w = d1w_ref[...]
    w0 = d0w[:, :C] + d0w[:, C:]
    d1a = d1w[:, :C]
    wf_ref[...] = lax.dot_general(
        d1a, w0, (((1,), (0,)), ((), ())),
        preferred_element_type=jnp.float32) + d1w[:, C:]
    bf_ref[...] = lax.dot_general(
        d1a, d0b_ref[...], (((1,), (0,)), ((), ())),
        preferred_element_type=jnp.float32) + d1b_ref[...]


def _stage1(f2, conv1_w, d0_w, d1_w, d0_b_col, d1_b_col):
    return pl.pallas_call(
        _k1_body,
        grid=(N // _TN1,),
        in_specs=[
            pl.BlockSpec((_TN1, C), lambda i: (i, 0)),
            pl.BlockSpec((C, C), lambda i: (0, 0)),
            pl.BlockSpec((C, 2 * C), lambda i: (0, 0)),
            pl.BlockSpec((C, 2 * C), lambda i: (0, 0)),
            pl.BlockSpec((C, 1), lambda i: (0, 0)),
            pl.BlockSpec((C, 1), lambda i: (0, 0)),
        ],
        out_specs=[
            pl.BlockSpec((_TN1, C), lambda i: (i, 0)),
            pl.BlockSpec((C, C), lambda i: (0, 0)),
            pl.BlockSpec((C, 1), lambda i: (0, 0)),
        ],
        out_shape=[
            jax.ShapeDtypeStruct((N, C), jnp.float32),
            jax.ShapeDtypeStruct((C, C), jnp.float32),
            jax.ShapeDtypeStruct((C, 1), jnp.float32),
        ],
    )(f2, conv1_w, d0_w, d1_w, d0_b_col, d1_b_col)


_NC = 2
_NS = 16
_NW = _NC * _NS
_PW = _EC // _NW
_CH = 128
_NFULL = _PW // _CH
_TAIL = _PW - _NFULL * _CH


def _sc_gather_body(g_hbm, idx_hbm, out_hbm, idx_v, rows_v, tail_v,
                    insem0, insem1, outsem0, outsem1):
    wid = lax.axis_index("s") * _NC + lax.axis_index("c")
    wbase = pl.multiple_of(wid * _PW, _PW)
    insem = (insem0, insem1)
    outsem = (outsem0, outsem1)

    pltpu.sync_copy(idx_hbm.at[pl.ds(wbase, _PW)], idx_v)

    def gather_descr(t, slot):
        off = pl.multiple_of(t * _CH, _CH)
        return pltpu.make_async_copy(
            g_hbm.at[idx_v.at[pl.ds(off, _CH)]], rows_v.at[slot], insem[slot])

    def wb_descr(t, slot):
        off = pl.multiple_of(wbase + t * _CH, _CH)
        return pltpu.make_async_copy(
            rows_v.at[slot], out_hbm.at[pl.ds(off, _CH)], outsem[slot])

    def start(t, slot):
        @pl.when(t < _NFULL)
        def _():
            gather_descr(t, slot).start()

    def drain(t, slot):
        gather_descr(t, slot).wait()
        wb_descr(t, slot).start()

    def wait_out(t, slot):
        wb_descr(t, slot).wait()

    start(0, 0)

    def body(tt, _):
        for b in range(2):
            t = 2 * tt + b
            nslot = 1 - b
            @pl.when(t >= 1)
            def _():
                wait_out(t - 1, nslot)
            start(t + 1, nslot)
            drain(t, b)
        return 0

    lax.fori_loop(0, _NFULL // 2, body, 0)
    if _NFULL % 2 == 1:
        t = _NFULL - 1
        wait_out(t - 1, 1 - (t % 2))
        drain(t, t % 2)
    wait_out(_NFULL - 1, (_NFULL - 1) % 2)

    if _TAIL:
        pltpu.make_async_copy(
            g_hbm.at[idx_v.at[pl.ds(_NFULL * _CH, _TAIL)]], tail_v.at[0],
            insem0).start()
        pltpu.make_async_copy(
            g_hbm.at[idx_v.at[pl.ds(_NFULL * _CH, _TAIL)]], tail_v.at[0],
            insem0).wait()
        pltpu.sync_copy(tail_v.at[0],
                        out_hbm.at[pl.ds(wbase + _NFULL * _CH, _TAIL)])


def _stage2(g, idx_chunk):
    mesh = plsc.VectorSubcoreMesh(core_axis_name="c", subcore_axis_name="s")
    run = functools.partial(
        pl.kernel,
        mesh=mesh,
        out_type=jax.ShapeDtypeStruct((_EC, C), jnp.float32),
        scratch_types=[
            pltpu.VMEM((_PW,), jnp.int32),
            pltpu.VMEM((2, _CH, C), jnp.float32),
            pltpu.VMEM((1, max(_TAIL, 8), C), jnp.float32),
            pltpu.SemaphoreType.DMA,
            pltpu.SemaphoreType.DMA,
            pltpu.SemaphoreType.DMA,
            pltpu.SemaphoreType.DMA,
        ],
    )(_sc_gather_body)
    return run(g, idx_chunk)


_TN3 = 200
_TE3 = _TN3 * K
_S3 = _NP // _TN3


def _k3_body(gath_ref, f_ref, c1w_ref, c2w_ref, bsum_ref, wf_ref,
             bf_ref, out_ref):
    f_blk = f_ref[...]
    w12 = c2w_ref[...] - c1w_ref[...]
    base = lax.dot_general(
        f_blk, w12, (((1,), (1,)), ((), ())),
        preferred_element_type=jnp.float32) + bsum_ref[...]
    g3 = gath_ref[...].reshape(_TN3, K, C)
    y = jnp.maximum(g3 + base[:, None, :], 0.0).reshape(_TE3, C)
    zt = lax.dot_general(
        wf_ref[...], y, (((1,), (1,)), ((), ())),
        preferred_element_type=jnp.float32)
    out_ref[...] = (zt + bf_ref[...]).reshape(C, _TN3, K)


def _stage3(p, buf, gathered_p, f2, conv1_w, conv2_w, bsum, wf, bf_col):
    data_specs = [
        pl.BlockSpec((_TE3, C), lambda i: (i, 0)),
        pl.BlockSpec((_TN3, C), lambda i, p=p: (p * _S3 + i, 0)),
        pl.BlockSpec((C, C), lambda i: (0, 0)),
        pl.BlockSpec((C, C), lambda i: (0, 0)),
        pl.BlockSpec((1, C), lambda i: (0, 0)),
        pl.BlockSpec((C, C), lambda i: (0, 0)),
        pl.BlockSpec((C, 1), lambda i: (0, 0)),
    ]
    data = (gathered_p, f2, conv1_w, conv2_w, bsum, wf, bf_col)
    if buf is None:
        in_specs, args, aliases, body = data_specs, data, {}, _k3_body
    else:
        def body(buf_ref, *rest):
            _k3_body(*rest)
        in_specs = [pl.BlockSpec(memory_space=pl.ANY)] + data_specs
        args = (buf,) + data
        aliases = {0: 0}
    return pl.pallas_call(
        body,
        grid=(_S3,),
        in_specs=in_specs,
        out_specs=pl.BlockSpec((C, _TN3, K), lambda i, p=p: (0, p * _S3 + i, 0)),
        out_shape=jax.ShapeDtypeStruct((C, N, K), jnp.float32),
        input_output_aliases=aliases,
    )(*args)


def kernel(f, k, idx, conv1_w, conv1_b, conv2_w, conv2_b,
           d0_w, d0_b, d1_w, d1_b):
    f2 = f.reshape(N, C)
    idx_flat = idx.reshape(E).astype(jnp.int32)
    bsum = (conv1_b + conv2_b).reshape(1, C)

    g, wf, bf_col = _stage1(f2, conv1_w, d0_w, d1_w,
                            d0_b.reshape(C, 1), d1_b.reshape(C, 1))

    gathered = [_stage2(g, lax.slice(idx_flat, (p * _EC,), ((p + 1) * _EC,)))
                for p in range(_P)]

    buf = None
    for p in range(_P):
        buf = _stage3(p, buf, gathered[p], f2, conv1_w, conv2_w,
                      bsum, wf, bf_col)
    return buf[None], idx


# speedup vs baseline: 2.4053x; 2.4053x over previous
import functools

import jax
import jax.numpy as jnp
from jax import lax
from jax.experimental import pallas as pl
from jax.experimental.pallas import tpu as pltpu
from jax.experimental.pallas import tpu_sc as plsc

N = 10000
K = 32
C = 128
E = N * K

_P = 5
_EC = E // _P
_NP = N // _P


_TN1 = 1000


def _k1_body(f_ref, c1w_ref, d0w_ref, d1w_ref, d0b_ref, d1b_ref,
             g_ref, wf_ref, bf_ref):
    f_blk = f_ref[...]
    g_ref[...] = lax.dot_general(
        f_blk, c1w_ref[...], (((1,), (1,)), ((), ())),
        preferred_element_type=jnp.float32)
    d0w = d0w_ref[...]
    d1w = d1w_ref[...]
    w0 = d0w[:, :C] + d0w[:, C:]
    d1a = d1w[:, :C]
    wf_ref[...] = lax.dot_general(
        d1a, w0, (((1,), (0,)), ((), ())),
        preferred_element_type=jnp.float32) + d1w[:, C:]
    bf_ref[...] = lax.dot_general(
        d0b_ref[...], d1a, (((1,), (1,)), ((), ())),
        preferred_element_type=jnp.float32) + d1b_ref[...]


def _stage1(f2, conv1_w, d0_w, d1_w, d0_b_col, d1_b_col):
    return pl.pallas_call(
        _k1_body,
        grid=(N // _TN1,),
        in_specs=[
            pl.BlockSpec((_TN1, C), lambda i: (i, 0)),
            pl.BlockSpec((C, C), lambda i: (0, 0)),
            pl.BlockSpec((C, 2 * C), lambda i: (0, 0)),
            pl.BlockSpec((C, 2 * C), lambda i: (0, 0)),
            pl.BlockSpec((1, C), lambda i: (0, 0)),
            pl.BlockSpec((1, C), lambda i: (0, 0)),
        ],
        out_specs=[
            pl.BlockSpec((_TN1, C), lambda i: (i, 0)),
            pl.BlockSpec((C, C), lambda i: (0, 0)),
            pl.BlockSpec((1, C), lambda i: (0, 0)),
        ],
        out_shape=[
            jax.ShapeDtypeStruct((N, C), jnp.float32),
            jax.ShapeDtypeStruct((C, C), jnp.float32),
            jax.ShapeDtypeStruct((1, C), jnp.float32),
        ],
    )(f2, conv1_w, d0_w, d1_w, d0_b_col, d1_b_col)


_NC = 2
_NS = 16
_NW = _NC * _NS
_PW = _EC // _NW
_CH = 128
_NFULL = _PW // _CH
_TAIL = _PW - _NFULL * _CH


def _sc_gather_body(g_hbm, idx_hbm, out_hbm, idx_v, rows_v, tail_v,
                    insem0, insem1, outsem0, outsem1):
    wid = lax.axis_index("s") * _NC + lax.axis_index("c")
    wbase = pl.multiple_of(wid * _PW, _PW)
    insem = (insem0, insem1)
    outsem = (outsem0, outsem1)

    pltpu.sync_copy(idx_hbm.at[pl.ds(wbase, _PW)], idx_v)

    def gather_descr(t, slot):
        off = pl.multiple_of(t * _CH, _CH)
        return pltpu.make_async_copy(
            g_hbm.at[idx_v.at[pl.ds(off, _CH)]], rows_v.at[slot], insem[slot])

    def wb_descr(t, slot):
        off = pl.multiple_of(wbase + t * _CH, _CH)
        return pltpu.make_async_copy(
            rows_v.at[slot], out_hbm.at[pl.ds(off, _CH)], outsem[slot])

    def start(t, slot):
        @pl.when(t < _NFULL)
        def _():
            gather_descr(t, slot).start()

    def drain(t, slot):
        gather_descr(t, slot).wait()
        wb_descr(t, slot).start()

    def wait_out(t, slot):
        wb_descr(t, slot).wait()

    start(0, 0)

    def body(tt, _):
        for b in range(2):
            t = 2 * tt + b
            nslot = 1 - b
            @pl.when(t >= 1)
            def _():
                wait_out(t - 1, nslot)
            start(t + 1, nslot)
            drain(t, b)
        return 0

    lax.fori_loop(0, _NFULL // 2, body, 0)
    if _NFULL % 2 == 1:
        t = _NFULL - 1
        wait_out(t - 1, 1 - (t % 2))
        drain(t, t % 2)
    wait_out(_NFULL - 1, (_NFULL - 1) % 2)

    if _TAIL:
        pltpu.make_async_copy(
            g_hbm.at[idx_v.at[pl.ds(_NFULL * _CH, _TAIL)]], tail_v.at[0],
            insem0).start()
        pltpu.make_async_copy(
            g_hbm.at[idx_v.at[pl.ds(_NFULL * _CH, _TAIL)]], tail_v.at[0],
            insem0).wait()
        pltpu.sync_copy(tail_v.at[0],
                        out_hbm.at[pl.ds(wbase + _NFULL * _CH, _TAIL)])


def _stage2(g, idx_chunk):
    mesh = plsc.VectorSubcoreMesh(core_axis_name="c", subcore_axis_name="s")
    run = functools.partial(
        pl.kernel,
        mesh=mesh,
        out_type=jax.ShapeDtypeStruct((_EC, C), jnp.float32),
        scratch_types=[
            pltpu.VMEM((_PW,), jnp.int32),
            pltpu.VMEM((2, _CH, C), jnp.float32),
            pltpu.VMEM((1, max(_TAIL, 8), C), jnp.float32),
            pltpu.SemaphoreType.DMA,
            pltpu.SemaphoreType.DMA,
            pltpu.SemaphoreType.DMA,
            pltpu.SemaphoreType.DMA,
        ],
    )(_sc_gather_body)
    return run(g, idx_chunk)


_TN3 = 200
_TE3 = _TN3 * K
_S3 = _NP // _TN3


def _k3_body(gath_ref, f_ref, c1w_ref, c2w_ref, bsum_ref, wf_ref,
             bf_ref, out_ref):
    f_blk = f_ref[...]
    w12 = c2w_ref[...] - c1w_ref[...]
    base = lax.dot_general(
        f_blk, w12, (((1,), (1,)), ((), ())),
        preferred_element_type=jnp.float32) + bsum_ref[...]
    g3 = gath_ref[...].reshape(_TN3, K, C)
    y = jnp.maximum(g3 + base[:, None, :], 0.0).reshape(_TE3, C)
    z = lax.dot_general(
        y, wf_ref[...], (((1,), (1,)), ((), ())),
        preferred_element_type=jnp.float32)
    out_ref[...] = (z + bf_ref[...]).reshape(_TN3, K, C)


def _stage3(p, buf, gathered_p, f2, conv1_w, conv2_w, bsum, wf, bf_row):
    data_specs = [
        pl.BlockSpec((_TE3, C), lambda i: (i, 0)),
        pl.BlockSpec((_TN3, C), lambda i, p=p: (p * _S3 + i, 0)),
        pl.BlockSpec((C, C), lambda i: (0, 0)),
        pl.BlockSpec((C, C), lambda i: (0, 0)),
        pl.BlockSpec((1, C), lambda i: (0, 0)),
        pl.BlockSpec((C, C), lambda i: (0, 0)),
        pl.BlockSpec((1, C), lambda i: (0, 0)),
    ]
    data = (gathered_p, f2, conv1_w, conv2_w, bsum, wf, bf_row)
    if buf is None:
        in_specs, args, aliases, body = data_specs, data, {}, _k3_body
    else:
        def body(buf_ref, *rest):
            _k3_body(*rest)
        in_specs = [pl.BlockSpec(memory_space=pl.ANY)] + data_specs
        args = (buf,) + data
        aliases = {0: 0}
    return pl.pallas_call(
        body,
        grid=(_S3,),
        in_specs=in_specs,
        out_specs=pl.BlockSpec((_TN3, K, C),
                               lambda i, p=p: (p * _S3 + i, 0, 0)),
        out_shape=jax.ShapeDtypeStruct((N, K, C), jnp.float32),
        input_output_aliases=aliases,
    )(*args)


def kernel(f, k, idx, conv1_w, conv1_b, conv2_w, conv2_b,
           d0_w, d0_b, d1_w, d1_b):
    f2 = f.reshape(N, C)
    idx_flat = idx.reshape(E).astype(jnp.int32)
    bsum = (conv1_b + conv2_b).reshape(1, C)

    g, wf, bf_row = _stage1(f2, conv1_w, d0_w, d1_w,
                            d0_b.reshape(1, C), d1_b.reshape(1, C))

    gathered = [_stage2(g, lax.slice(idx_flat, (p * _EC,), ((p + 1) * _EC,)))
                for p in range(_P)]

    buf = None
    for p in range(_P):
        buf = _stage3(p, buf, gathered[p], f2, conv1_w, conv2_w,
                      bsum, wf, bf_row)
    return jnp.transpose(buf, (2, 0, 1))[None], idx
